# baseline (speedup 1.0000x reference)
import functools

import jax
import jax.numpy as jnp
from jax import lax
from jax.experimental import pallas as pl
from jax.experimental.pallas import tpu as pltpu
from jax.experimental.pallas import tpu_sc as plsc

NUM_EMB = 1_000_000
D = 64
BATCH = 16384
HIST = 50
NC = 2
NS = 16
NW = NC * NS
BPW = BATCH // NW
UPW = 4
NUNITS = HIST * UPW

_mesh = plsc.VectorSubcoreMesh(core_axis_name="c", subcore_axis_name="s")


@functools.partial(
    pl.kernel,
    mesh=_mesh,
    compiler_params=pltpu.CompilerParams(
        use_tc_tiling_on_sc=False, needs_layout_passes=False
    ),
    out_type=jax.ShapeDtypeStruct((HIST, 8, 128, 8, 128), jnp.float32),
    scratch_types=[
        pltpu.VMEM((BPW, HIST), jnp.int32),
        pltpu.VMEM((HIST, BPW), jnp.int32),
        pltpu.VMEM((2, 128, D), jnp.float32),
        pltpu.VMEM((2, D, 128), jnp.float32),
    ]
    + [pltpu.SemaphoreType.DMA] * 4,
)
def _emb5(x_hbm, table_hbm, out_hbm, idx_v, idxt_v, rows_v, trans_v, *sems):
    gsems, ssems = sems[:2], sems[2:]
    wid = lax.axis_index("s") * NC + lax.axis_index("c")
    ii = lax.broadcasted_iota(jnp.int32, (16,), 0)

    pltpu.sync_copy(x_hbm.at[pl.ds(wid * BPW, BPW)], idx_v)

    def tr_idx(h, carry):
        col = jnp.full((16,), h, jnp.int32)
        for t in range(BPW // 16):
            v = plsc.load_gather(idx_v, [ii + 16 * t, col])
            idxt_v[h, pl.ds(16 * t, 16)] = v
        return carry

    lax.fori_loop(0, HIST, tr_idx, 0)

    def issue_gather(s, buf):
        h = lax.shift_right_logical(s, 2)
        u = jnp.bitwise_and(s, 3)
        pltpu.async_copy(
            table_hbm.at[idxt_v.at[h, pl.ds(128 * u, 128)]],
            rows_v.at[buf],
            gsems[buf],
        )

    def wait_gather(buf):
        pltpu.make_async_copy(
            table_hbm.at[idxt_v.at[0, pl.ds(0, 128)]], rows_v.at[buf], gsems[buf]
        ).wait()

    def transpose(buf):
        def body(c, carry):
            col = jnp.full((16,), c, jnp.int32)
            for t in range(8):
                v = plsc.load_gather(rows_v.at[buf], [ii + 16 * t, col])
                trans_v[buf, c, pl.ds(16 * t, 16)] = v
            return carry

        lax.fori_loop(0, D, body, 0)

    def issue_stores(s, buf):
        h = lax.shift_right_logical(s, 2)
        u = jnp.bitwise_and(s, 3)
        bh = wid * UPW + u
        for ch in range(8):
            pltpu.async_copy(
                trans_v.at[buf, pl.ds(8 * ch, 8)],
                out_hbm.at[h, ch, bh],
                ssems[buf],
            )

    def wait_stores(buf):
        for ch in range(8):
            pltpu.make_async_copy(
                trans_v.at[buf, pl.ds(0, 8)], out_hbm.at[0, 0, 0], ssems[buf]
            ).wait()

    def slot(s, buf):
        @pl.when(s + 1 < NUNITS)
        def _():
            issue_gather(s + 1, 1 - buf)

        wait_gather(buf)

        @pl.when(s >= 2)
        def _():
            wait_stores(buf)

        transpose(buf)
        issue_stores(s, buf)

    issue_gather(jnp.int32(0), 0)

    def grp(g, carry):
        slot(2 * g, 0)
        slot(2 * g + 1, 1)
        return carry

    lax.fori_loop(0, NUNITS // 2, grp, 0)
    wait_stores(0)
    wait_stores(1)


def kernel(x, weight):
    out5 = _emb5(x.astype(jnp.int32), weight)
    return out5.transpose(2, 4, 0, 1, 3).reshape(BATCH, HIST, D)
